# baseline (speedup 1.0000x reference)
import jax
import jax.numpy as jnp
from jax.experimental import pallas as pl
from jax.experimental.pallas import tpu as pltpu

E_TOT = 32 * 1024
DIM = 128
N_CHUNKS = 1
CHUNK = E_TOT // N_CHUNKS


def _copy_kernel(e_hbm, o_hbm, buf, in_sems, out_sems):
    for i in range(N_CHUNKS):
        pltpu.make_async_copy(
            e_hbm.at[pl.ds(i * CHUNK, CHUNK), :], buf.at[i], in_sems.at[i]
        ).start()
    for i in range(N_CHUNKS):
        pltpu.make_async_copy(
            e_hbm.at[pl.ds(i * CHUNK, CHUNK), :], buf.at[i], in_sems.at[i]
        ).wait()
        pltpu.make_async_copy(
            buf.at[i], o_hbm.at[pl.ds(i * CHUNK, CHUNK), :], out_sems.at[i]
        ).start()
    for i in range(N_CHUNKS):
        pltpu.make_async_copy(
            buf.at[i], o_hbm.at[pl.ds(i * CHUNK, CHUNK), :], out_sems.at[i]
        ).wait()


def kernel(h, e, params, edge_index):
    out = pl.pallas_call(
        _copy_kernel,
        in_specs=[pl.BlockSpec(memory_space=pltpu.MemorySpace.HBM)],
        out_specs=pl.BlockSpec(memory_space=pltpu.MemorySpace.HBM),
        out_shape=jax.ShapeDtypeStruct((E_TOT, DIM), e.dtype),
        scratch_shapes=[
            pltpu.VMEM((N_CHUNKS, CHUNK, DIM), jnp.float32),
            pltpu.SemaphoreType.DMA((N_CHUNKS,)),
            pltpu.SemaphoreType.DMA((N_CHUNKS,)),
        ],
    )(e)
    return out


# speedup vs baseline: 1.0702x; 1.0702x over previous
import jax
import jax.numpy as jnp
from jax.experimental import pallas as pl
from jax.experimental.pallas import tpu as pltpu

E_TOT = 32 * 1024
DIM = 128
N_CHUNKS = 2
CHUNK = E_TOT // N_CHUNKS


def _copy_kernel(e_hbm, o_hbm, buf, in_sems, out_sems):
    for i in range(N_CHUNKS):
        pltpu.make_async_copy(
            e_hbm.at[pl.ds(i * CHUNK, CHUNK), :], buf.at[i], in_sems.at[i]
        ).start()
    for i in range(N_CHUNKS):
        pltpu.make_async_copy(
            e_hbm.at[pl.ds(i * CHUNK, CHUNK), :], buf.at[i], in_sems.at[i]
        ).wait()
        pltpu.make_async_copy(
            buf.at[i], o_hbm.at[pl.ds(i * CHUNK, CHUNK), :], out_sems.at[i]
        ).start()
    for i in range(N_CHUNKS):
        pltpu.make_async_copy(
            buf.at[i], o_hbm.at[pl.ds(i * CHUNK, CHUNK), :], out_sems.at[i]
        ).wait()


def kernel(h, e, params, edge_index):
    out = pl.pallas_call(
        _copy_kernel,
        in_specs=[pl.BlockSpec(memory_space=pltpu.MemorySpace.HBM)],
        out_specs=pl.BlockSpec(memory_space=pltpu.MemorySpace.HBM),
        out_shape=jax.ShapeDtypeStruct((E_TOT, DIM), e.dtype),
        scratch_shapes=[
            pltpu.VMEM((N_CHUNKS, CHUNK, DIM), jnp.float32),
            pltpu.SemaphoreType.DMA((N_CHUNKS,)),
            pltpu.SemaphoreType.DMA((N_CHUNKS,)),
        ],
    )(e)
    return out
